# baseline (speedup 1.0000x reference)
import jax
import jax.numpy as jnp
from jax.experimental import pallas as pl

H = 12


def _fused_attn_ln(t_ref, nb_ref, sw_ref, wqt_ref, wk3_ref, wv3t_ref,
                   wot_ref, scale_ref, gamma_ref, beta_ref, o_ref):
    t = t_ref[...]
    nb = nb_ref[...]
    Bb, D = t.shape
    hd = D // H

    q = jnp.dot(t, wqt_ref[...], preferred_element_type=jnp.float32)
    qh = q.reshape(Bb, H, hd)
    a = jnp.einsum('bhk,hkd->bhd', qh, wk3_ref[...],
                   preferred_element_type=jnp.float32)
    scores = jnp.einsum('bhd,bnd->bhn', a, nb,
                        preferred_element_type=jnp.float32) * (hd ** -0.5)
    scores = scores + scale_ref[0, 0] * sw_ref[...][:, None, :]

    m = jnp.max(scores, axis=-1, keepdims=True)
    e = jnp.exp(scores - m)
    w = e / jnp.sum(e, axis=-1, keepdims=True)

    msum = jnp.einsum('bhn,bnd->bhd', w, nb,
                      preferred_element_type=jnp.float32)
    ao = jnp.einsum('bhd,hdk->bhk', msum, wv3t_ref[...],
                    preferred_element_type=jnp.float32).reshape(Bb, D)

    y = t + jnp.dot(ao, wot_ref[...], preferred_element_type=jnp.float32)
    mu = jnp.mean(y, axis=-1, keepdims=True)
    yc = y - mu
    var = jnp.mean(yc * yc, axis=-1, keepdims=True)
    o_ref[...] = yc * jax.lax.rsqrt(var + 1e-5) * gamma_ref[...] + beta_ref[...]


def kernel(target_emb, neighbor_embs, swing_scores, Wq, Wk, Wv, Wo,
           swing_scale, ln_gamma, ln_beta):
    B, D = target_emb.shape
    N = neighbor_embs.shape[1]
    hd = D // H
    Bb = 32
    grid = (B // Bb,)

    wqt = Wq.T
    wk3 = Wk.reshape(H, hd, D)
    wv3t = Wv.reshape(H, hd, D).transpose(0, 2, 1)
    wot = Wo.T
    scale2 = swing_scale.reshape(1, 1)
    gamma2 = ln_gamma.reshape(1, D)
    beta2 = ln_beta.reshape(1, D)

    return pl.pallas_call(
        _fused_attn_ln,
        grid=grid,
        in_specs=[
            pl.BlockSpec((Bb, D), lambda i: (i, 0)),
            pl.BlockSpec((Bb, N, D), lambda i: (i, 0, 0)),
            pl.BlockSpec((Bb, N), lambda i: (i, 0)),
            pl.BlockSpec((D, D), lambda i: (0, 0)),
            pl.BlockSpec((H, hd, D), lambda i: (0, 0, 0)),
            pl.BlockSpec((H, D, hd), lambda i: (0, 0, 0)),
            pl.BlockSpec((D, D), lambda i: (0, 0)),
            pl.BlockSpec((1, 1), lambda i: (0, 0)),
            pl.BlockSpec((1, D), lambda i: (0, 0)),
            pl.BlockSpec((1, D), lambda i: (0, 0)),
        ],
        out_specs=pl.BlockSpec((Bb, D), lambda i: (i, 0)),
        out_shape=jax.ShapeDtypeStruct((B, D), jnp.float32),
    )(target_emb, neighbor_embs, swing_scores, wqt, wk3, wv3t, wot,
      scale2, gamma2, beta2)


# speedup vs baseline: 1.3943x; 1.3943x over previous
import jax
import jax.numpy as jnp
from jax.experimental import pallas as pl

H = 12


def kernel(target_emb, neighbor_embs, swing_scores, Wq, Wk, Wv, Wo,
           swing_scale, ln_gamma, ln_beta):
    B, D = target_emb.shape
    N = neighbor_embs.shape[1]
    hd = D // H
    Bb = 64
    SB = 8
    nsub = Bb // SB
    f32 = jnp.float32

    def _fused(t_ref, nb_ref, sw_ref, wqt_ref, wk3_ref, wv3t_ref, wot_ref,
               scale_ref, gamma_ref, beta_ref, o_ref):
        t = t_ref[...]
        q = jnp.dot(t, wqt_ref[...], preferred_element_type=f32)
        a_heads = [jnp.dot(q[:, h * hd:(h + 1) * hd], wk3_ref[h],
                           preferred_element_type=f32) for h in range(H)]
        scale = scale_ref[0, 0]
        lane = jax.lax.broadcasted_iota(jnp.int32, (SB, SB * N), 1)
        row = jax.lax.broadcasted_iota(jnp.int32, (SB, SB * N), 0)
        sel_full = jnp.tile((lane // N) == row, (H, 1))

        m_pieces = []
        for s in range(nsub):
            nb_s = nb_ref[pl.ds(s * SB * N, SB * N), :]
            a_sub = jnp.concatenate(
                [a_heads[h][s * SB:(s + 1) * SB] for h in range(H)],
                axis=0)
            scores = jax.lax.dot_general(
                a_sub, nb_s, (((1,), (1,)), ((), ())),
                preferred_element_type=f32) * (hd ** -0.5)
            sw_s = sw_ref[s * SB:(s + 1) * SB, :]
            bias = scale * jnp.tile(sw_s, (1, SB))
            scores = scores + jnp.tile(bias, (H, 1))
            scores = jnp.where(sel_full, scores, -1e30)
            mx = jnp.max(scores, axis=1, keepdims=True)
            e = jnp.exp(scores - mx)
            w = e / jnp.sum(e, axis=1, keepdims=True)
            m_pieces.append(jnp.dot(w, nb_s,
                                    preferred_element_type=f32))

        ao_parts = []
        for h in range(H):
            m_h = jnp.concatenate(
                [m_pieces[s][h * SB:(h + 1) * SB] for s in range(nsub)],
                axis=0)
            ao_parts.append(jnp.dot(m_h, wv3t_ref[h],
                                    preferred_element_type=f32))
        ao = jnp.concatenate(ao_parts, axis=1)

        y = t + jnp.dot(ao, wot_ref[...], preferred_element_type=f32)
        mu = jnp.mean(y, axis=-1, keepdims=True)
        yc = y - mu
        var = jnp.mean(yc * yc, axis=-1, keepdims=True)
        o_ref[...] = (yc * jax.lax.rsqrt(var + 1e-5) * gamma_ref[...]
                      + beta_ref[...])

    nbf = neighbor_embs.reshape(B * N, D)
    wqt = Wq.T
    wk3 = Wk.reshape(H, hd, D)
    wv3t = Wv.reshape(H, hd, D).transpose(0, 2, 1)
    wot = Wo.T
    scale2 = swing_scale.reshape(1, 1)
    gamma2 = ln_gamma.reshape(1, D)
    beta2 = ln_beta.reshape(1, D)

    return pl.pallas_call(
        _fused,
        grid=(B // Bb,),
        in_specs=[
            pl.BlockSpec((Bb, D), lambda i: (i, 0)),
            pl.BlockSpec((Bb * N, D), lambda i: (i, 0)),
            pl.BlockSpec((Bb, N), lambda i: (i, 0)),
            pl.BlockSpec((D, D), lambda i: (0, 0)),
            pl.BlockSpec((H, hd, D), lambda i: (0, 0, 0)),
            pl.BlockSpec((H, D, hd), lambda i: (0, 0, 0)),
            pl.BlockSpec((D, D), lambda i: (0, 0)),
            pl.BlockSpec((1, 1), lambda i: (0, 0)),
            pl.BlockSpec((1, D), lambda i: (0, 0)),
            pl.BlockSpec((1, D), lambda i: (0, 0)),
        ],
        out_specs=pl.BlockSpec((Bb, D), lambda i: (i, 0)),
        out_shape=jax.ShapeDtypeStruct((B, D), jnp.float32),
    )(target_emb, nbf, swing_scores, wqt, wk3, wv3t, wot,
      scale2, gamma2, beta2)


# speedup vs baseline: 2.0387x; 1.4621x over previous
import jax
import jax.numpy as jnp
from jax.experimental import pallas as pl

H = 12


def kernel(target_emb, neighbor_embs, swing_scores, Wq, Wk, Wv, Wo,
           swing_scale, ln_gamma, ln_beta):
    B, D = target_emb.shape
    N = neighbor_embs.shape[1]
    hd = D // H
    Bb = 64
    SB = 8
    nsub = Bb // SB
    f32 = jnp.float32

    def _fused(t_ref, nb_ref, sw_ref, wqt_ref, wk3_ref, wv3t_ref, wot_ref,
               scale_ref, gamma_ref, beta_ref, o_ref):
        t = t_ref[...]
        q = jnp.dot(t, wqt_ref[...], preferred_element_type=f32)
        a_heads = [jnp.dot(q[:, h * hd:(h + 1) * hd], wk3_ref[h],
                           preferred_element_type=f32) for h in range(H)]
        scale = scale_ref[0, 0]
        col = jax.lax.broadcasted_iota(jnp.int32, (SB, 1, H * SB), 2)
        row = jax.lax.broadcasted_iota(jnp.int32, (SB, 1, H * SB), 0)
        own = (col % SB) == row

        m_pieces = []
        for s in range(nsub):
            nb_s = nb_ref[s * SB:(s + 1) * SB]
            a_sub = jnp.concatenate(
                [a_heads[h][s * SB:(s + 1) * SB] for h in range(H)],
                axis=0)
            scores = jax.lax.dot_general(
                nb_s, a_sub, (((2,), (1,)), ((), ())),
                preferred_element_type=f32) * (hd ** -0.5)
            sw_s = sw_ref[s * SB:(s + 1) * SB, :]
            scores = scores + (scale * sw_s)[:, :, None]
            mx = jnp.max(scores, axis=1, keepdims=True)
            e = jnp.exp(scores - mx)
            w = e / jnp.sum(e, axis=1, keepdims=True)
            wm = jnp.where(own, w, 0.0)
            acc = None
            for b in range(SB):
                p = jax.lax.dot_general(
                    wm[b], nb_s[b], (((0,), (0,)), ((), ())),
                    preferred_element_type=f32)
                acc = p if acc is None else acc + p
            m_pieces.append(acc)

        ao_parts = []
        for h in range(H):
            m_h = jnp.concatenate(
                [m_pieces[s][h * SB:(h + 1) * SB] for s in range(nsub)],
                axis=0)
            ao_parts.append(jnp.dot(m_h, wv3t_ref[h],
                                    preferred_element_type=f32))
        ao = jnp.concatenate(ao_parts, axis=1)

        y = t + jnp.dot(ao, wot_ref[...], preferred_element_type=f32)
        mu = jnp.mean(y, axis=-1, keepdims=True)
        yc = y - mu
        var = jnp.mean(yc * yc, axis=-1, keepdims=True)
        o_ref[...] = (yc * jax.lax.rsqrt(var + 1e-5) * gamma_ref[...]
                      + beta_ref[...])

    wqt = Wq.T
    wk3 = Wk.reshape(H, hd, D)
    wv3t = Wv.reshape(H, hd, D).transpose(0, 2, 1)
    wot = Wo.T
    scale2 = swing_scale.reshape(1, 1)
    gamma2 = ln_gamma.reshape(1, D)
    beta2 = ln_beta.reshape(1, D)

    return pl.pallas_call(
        _fused,
        grid=(B // Bb,),
        in_specs=[
            pl.BlockSpec((Bb, D), lambda i: (i, 0)),
            pl.BlockSpec((Bb, N, D), lambda i: (i, 0, 0)),
            pl.BlockSpec((Bb, N), lambda i: (i, 0)),
            pl.BlockSpec((D, D), lambda i: (0, 0)),
            pl.BlockSpec((H, hd, D), lambda i: (0, 0, 0)),
            pl.BlockSpec((H, D, hd), lambda i: (0, 0, 0)),
            pl.BlockSpec((D, D), lambda i: (0, 0)),
            pl.BlockSpec((1, 1), lambda i: (0, 0)),
            pl.BlockSpec((1, D), lambda i: (0, 0)),
            pl.BlockSpec((1, D), lambda i: (0, 0)),
        ],
        out_specs=pl.BlockSpec((Bb, D), lambda i: (i, 0)),
        out_shape=jax.ShapeDtypeStruct((B, D), jnp.float32),
    )(target_emb, neighbor_embs, swing_scores, wqt, wk3, wv3t, wot,
      scale2, gamma2, beta2)


# speedup vs baseline: 2.0429x; 1.0021x over previous
import jax
import jax.numpy as jnp
from jax.experimental import pallas as pl
from jax.experimental.pallas import tpu as pltpu

H = 12


def kernel(target_emb, neighbor_embs, swing_scores, Wq, Wk, Wv, Wo,
           swing_scale, ln_gamma, ln_beta):
    B, D = target_emb.shape
    N = neighbor_embs.shape[1]
    hd = D // H
    Bb = 64
    SB = 8
    nsub = Bb // SB
    f32 = jnp.float32

    def _fused(t_ref, nb_ref, sw_ref, wqt_ref, wk3_ref, wv3t_ref, wot_ref,
               scale_ref, gamma_ref, beta_ref, o_ref):
        t = t_ref[...]
        q = jnp.dot(t, wqt_ref[...], preferred_element_type=f32)
        a_heads = [jnp.dot(q[:, h * hd:(h + 1) * hd], wk3_ref[h],
                           preferred_element_type=f32) for h in range(H)]
        scale = scale_ref[0, 0]
        col = jax.lax.broadcasted_iota(jnp.int32, (SB, 1, H * SB), 2)
        row = jax.lax.broadcasted_iota(jnp.int32, (SB, 1, H * SB), 0)
        own = (col % SB) == row

        m_pieces = []
        for s in range(nsub):
            nb_s = nb_ref[s * SB:(s + 1) * SB]
            a_sub = jnp.concatenate(
                [a_heads[h][s * SB:(s + 1) * SB] for h in range(H)],
                axis=0)
            scores = jax.lax.dot_general(
                nb_s, a_sub, (((2,), (1,)), ((), ())),
                preferred_element_type=f32) * (hd ** -0.5)
            sw_s = sw_ref[s * SB:(s + 1) * SB, :]
            scores = scores + (scale * sw_s)[:, :, None]
            mx = jnp.max(scores, axis=1, keepdims=True)
            e = jnp.exp(scores - mx)
            w = e / jnp.sum(e, axis=1, keepdims=True)
            wm = jnp.where(own, w, 0.0)
            acc = None
            for b in range(SB):
                p = jax.lax.dot_general(
                    wm[b], nb_s[b], (((0,), (0,)), ((), ())),
                    preferred_element_type=f32)
                acc = p if acc is None else acc + p
            m_pieces.append(acc)

        ao_parts = []
        for h in range(H):
            m_h = jnp.concatenate(
                [m_pieces[s][h * SB:(h + 1) * SB] for s in range(nsub)],
                axis=0)
            ao_parts.append(jnp.dot(m_h, wv3t_ref[h],
                                    preferred_element_type=f32))
        ao = jnp.concatenate(ao_parts, axis=1)

        y = t + jnp.dot(ao, wot_ref[...], preferred_element_type=f32)
        mu = jnp.mean(y, axis=-1, keepdims=True)
        yc = y - mu
        var = jnp.mean(yc * yc, axis=-1, keepdims=True)
        o_ref[...] = (yc * jax.lax.rsqrt(var + 1e-5) * gamma_ref[...]
                      + beta_ref[...])

    wqt = Wq.T
    wk3 = Wk.reshape(H, hd, D)
    wv3t = Wv.reshape(H, hd, D).transpose(0, 2, 1)
    wot = Wo.T
    scale2 = swing_scale.reshape(1, 1)
    gamma2 = ln_gamma.reshape(1, D)
    beta2 = ln_beta.reshape(1, D)

    return pl.pallas_call(
        _fused,
        grid=(B // Bb,),
        in_specs=[
            pl.BlockSpec((Bb, D), lambda i: (i, 0)),
            pl.BlockSpec((Bb, N, D), lambda i: (i, 0, 0)),
            pl.BlockSpec((Bb, N), lambda i: (i, 0)),
            pl.BlockSpec((D, D), lambda i: (0, 0)),
            pl.BlockSpec((H, hd, D), lambda i: (0, 0, 0)),
            pl.BlockSpec((H, D, hd), lambda i: (0, 0, 0)),
            pl.BlockSpec((D, D), lambda i: (0, 0)),
            pl.BlockSpec((1, 1), lambda i: (0, 0)),
            pl.BlockSpec((1, D), lambda i: (0, 0)),
            pl.BlockSpec((1, D), lambda i: (0, 0)),
        ],
        out_specs=pl.BlockSpec((Bb, D), lambda i: (i, 0)),
        out_shape=jax.ShapeDtypeStruct((B, D), jnp.float32),
        compiler_params=pltpu.CompilerParams(
            vmem_limit_bytes=120 * 1024 * 1024),
    )(target_emb, neighbor_embs, swing_scores, wqt, wk3, wv3t, wot,
      scale2, gamma2, beta2)
